# Initial kernel scaffold; baseline (speedup 1.0000x reference)
import jax
import jax.numpy as jnp
from jax.experimental import pallas as pl


def kernel(x, W, b_enc, b_dec):
    raise NotImplementedError("write your pallas kernel here")


# speedup vs baseline: 2.5039x; 2.5039x over previous
import functools

import jax
import jax.numpy as jnp
from jax.experimental import pallas as pl

_INPUT_DIM = 4096
_HIDDEN_DIM = 16384
_K = 256
_SPARSITY_COEF = 0.001
_BH_ENC = 1024
_BH_DEC = 1024


def _encode_body(x_ref, w_ref, b_ref, e_ref):
    acc = jax.lax.dot_general(
        x_ref[...], w_ref[...], (((1,), (1,)), ((), ())),
        preferred_element_type=jnp.float32)
    e_ref[...] = jnp.clip(acc + b_ref[...], -10.0, 10.0)


def _topk_body(e_ref, lat_ref, ssum_ref):
    e = e_ref[...]
    i32 = jax.lax.bitcast_convert_type(e, jnp.int32)
    neg = jnp.int32(-2147483648)
    key = jnp.where(i32 >= 0, i32, neg - i32 - 1)
    lo = jnp.min(key, axis=1, keepdims=True)
    hi = jnp.max(key, axis=1, keepdims=True) + 1

    def body(_, carry):
        lo, hi = carry
        mid = jnp.bitwise_and(lo, hi) + jnp.right_shift(jnp.bitwise_xor(lo, hi), 1)
        cnt = jnp.sum((key >= mid).astype(jnp.int32), axis=1, keepdims=True)
        pred = cnt >= _K
        return jnp.where(pred, mid, lo), jnp.where(pred, hi, mid)

    lo, hi = jax.lax.fori_loop(0, 32, body, (lo, hi))
    latent = jnp.where(key >= lo, jnp.maximum(e, 0.0), 0.0)
    lat_ref[...] = latent
    ssum_ref[...] = jnp.sum(latent).reshape(1, 1)


def _decode_body(lat_ref, w_ref, x_ref, bd_ref, ssum_ref, rec_ref, loss_ref,
                 *, nsteps, batch):
    i = pl.program_id(0)

    @pl.when(i == 0)
    def _():
        rec_ref[...] = jnp.broadcast_to(bd_ref[...], (batch, _INPUT_DIM))

    rec_ref[...] += jnp.dot(lat_ref[...], w_ref[...],
                            preferred_element_type=jnp.float32)

    @pl.when(i == nsteps - 1)
    def _():
        diff = rec_ref[...] - x_ref[...]
        recon_loss = jnp.minimum(jnp.mean(diff * diff), 100.0)
        sparsity = jnp.minimum(ssum_ref[...][0, 0] / (batch * _HIDDEN_DIM), 10.0)
        loss_ref[...] = (recon_loss + _SPARSITY_COEF * sparsity).reshape(1, 1)


def kernel(x, W, b_enc, b_dec):
    b, t, c = x.shape
    n = b * t
    x2 = x.reshape(n, c)
    be2 = b_enc.reshape(1, _HIDDEN_DIM)
    bd2 = b_dec.reshape(1, _INPUT_DIM)

    e = pl.pallas_call(
        _encode_body,
        grid=(_HIDDEN_DIM // _BH_ENC,),
        in_specs=[
            pl.BlockSpec((n, _INPUT_DIM), lambda i: (0, 0)),
            pl.BlockSpec((_BH_ENC, _INPUT_DIM), lambda i: (i, 0)),
            pl.BlockSpec((1, _BH_ENC), lambda i: (0, i)),
        ],
        out_specs=pl.BlockSpec((n, _BH_ENC), lambda i: (0, i)),
        out_shape=jax.ShapeDtypeStruct((n, _HIDDEN_DIM), jnp.float32),
    )(x2, W, be2)

    latent, ssum = pl.pallas_call(
        _topk_body,
        in_specs=[pl.BlockSpec((n, _HIDDEN_DIM), lambda: (0, 0))],
        out_specs=[
            pl.BlockSpec((n, _HIDDEN_DIM), lambda: (0, 0)),
            pl.BlockSpec((1, 1), lambda: (0, 0)),
        ],
        out_shape=[
            jax.ShapeDtypeStruct((n, _HIDDEN_DIM), jnp.float32),
            jax.ShapeDtypeStruct((1, 1), jnp.float32),
        ],
    )(e)

    nsteps = _HIDDEN_DIM // _BH_DEC
    rec, loss = pl.pallas_call(
        functools.partial(_decode_body, nsteps=nsteps, batch=n),
        grid=(nsteps,),
        in_specs=[
            pl.BlockSpec((n, _BH_DEC), lambda i: (0, i)),
            pl.BlockSpec((_BH_DEC, _INPUT_DIM), lambda i: (i, 0)),
            pl.BlockSpec((n, _INPUT_DIM), lambda i: (0, 0)),
            pl.BlockSpec((1, _INPUT_DIM), lambda i: (0, 0)),
            pl.BlockSpec((1, 1), lambda i: (0, 0)),
        ],
        out_specs=[
            pl.BlockSpec((n, _INPUT_DIM), lambda i: (0, 0)),
            pl.BlockSpec((1, 1), lambda i: (0, 0)),
        ],
        out_shape=[
            jax.ShapeDtypeStruct((n, _INPUT_DIM), jnp.float32),
            jax.ShapeDtypeStruct((1, 1), jnp.float32),
        ],
    )(latent, W, x2, bd2, ssum)

    return (rec.reshape(b, t, c), latent.reshape(b, t, _HIDDEN_DIM),
            loss[0, 0])
